# Initial kernel scaffold; baseline (speedup 1.0000x reference)
import jax
import jax.numpy as jnp
from jax.experimental import pallas as pl


def kernel(x, batch, W1, b1, W2, b2, Wp, bp):
    raise NotImplementedError("write your pallas kernel here")


# speedup vs baseline: 3.9398x; 3.9398x over previous
import functools

import jax
import jax.numpy as jnp
from jax import lax
from jax.experimental import pallas as pl
from jax.experimental.pallas import tpu as pltpu
from jax.experimental.pallas import tpu_sc as plsc

N = 320000
H = 128
G = 512

ROWS_PER_TILE = 1280
NUM_TILES = N // ROWS_PER_TILE


def _pool_body(x_ref, b_ref, w1_ref, b1_ref, w2_ref, b2_ref, wp_ref, bp_ref,
               ctx_ref, acc_ex, acc_e):
  i = pl.program_id(0)

  @pl.when(i == 0)
  def _init():
    acc_ex[...] = jnp.zeros_like(acc_ex)
    acc_e[...] = jnp.zeros_like(acc_e)

  xb = x_ref[...]
  bt = b_ref[...]
  w2 = w2_ref[...]

  h = jnp.tanh(
      lax.dot_general(xb, w1_ref[...], (((1,), (1,)), ((), ())),
                      preferred_element_type=jnp.float32)
      + b1_ref[...])
  s = jnp.sum(h * w2, axis=1, keepdims=True) + b2_ref[...]
  shift = jnp.sum(jnp.abs(w2)) + jnp.abs(b2_ref[0, 0])
  e = jnp.exp(s - shift)
  ex = xb * e

  g_first = bt[0, 0]
  g_last = bt[ROWS_PER_TILE - 1, 0]

  def seg_body(g, _):
    m = bt == g
    pe = jnp.sum(jnp.where(m, e, 0.0))
    pex = jnp.sum(jnp.where(m, ex, 0.0), axis=0)
    acc_ex[pl.ds(g, 1), :] += pex[None, :]
    acc_e[pl.ds(g, 1), :] += pe.reshape(1, 1)
    return 0

  lax.fori_loop(g_first, g_last + 1, seg_body, 0)

  @pl.when(i == NUM_TILES - 1)
  def _finish():
    se = acc_e[...]
    ge = acc_ex[...] / jnp.where(se > 0.0, se, 1.0)
    ctx = lax.dot_general(ge, wp_ref[...], (((1,), (1,)), ((), ())),
                          preferred_element_type=jnp.float32) + bp_ref[...]
    ctx_ref[...] = ctx


def _segment_context(x, batch2d, W1, b1, W2, b2, Wp, bp, interpret=False):
  T = ROWS_PER_TILE
  return pl.pallas_call(
      _pool_body,
      grid=(NUM_TILES,),
      in_specs=[
          pl.BlockSpec((T, H), lambda i: (i, 0)),
          pl.BlockSpec((T, 1), lambda i: (i, 0)),
          pl.BlockSpec((H // 2, H), lambda i: (0, 0)),
          pl.BlockSpec((1, H // 2), lambda i: (0, 0)),
          pl.BlockSpec((1, H // 2), lambda i: (0, 0)),
          pl.BlockSpec((1, 1), lambda i: (0, 0)),
          pl.BlockSpec((H, H), lambda i: (0, 0)),
          pl.BlockSpec((1, H), lambda i: (0, 0)),
      ],
      out_specs=pl.BlockSpec((G, H), lambda i: (0, 0)),
      out_shape=jax.ShapeDtypeStruct((G, H), jnp.float32),
      scratch_shapes=[
          pltpu.VMEM((G, H), jnp.float32),
          pltpu.VMEM((G, 1), jnp.float32),
      ],
      compiler_params=pltpu.CompilerParams(
          dimension_semantics=("arbitrary",)),
      interpret=interpret,
  )(x, batch2d, W1, b1, W2, b2, Wp, bp)


CHUNK = 128
NUM_CHUNKS = N // CHUNK
NUM_WORKERS = 32
BASE_CHUNKS = NUM_CHUNKS // NUM_WORKERS
EXTRA = NUM_CHUNKS - BASE_CHUNKS * NUM_WORKERS


def _gather_body(ctx_hbm, idx_hbm, out_hbm, idx_v, rows_v, sem):
  wid = lax.axis_index("s") * 2 + lax.axis_index("c")
  start = wid * BASE_CHUNKS + jnp.minimum(wid, EXTRA)
  count = BASE_CHUNKS + jnp.where(wid < EXTRA, 1, 0)

  def chunk_body(j, _):
    base = (start + j) * CHUNK
    pltpu.sync_copy(idx_hbm.at[pl.ds(base, CHUNK)], idx_v)
    pltpu.async_copy(ctx_hbm.at[idx_v], rows_v, sem).wait()
    pltpu.sync_copy(rows_v, out_hbm.at[pl.ds(base, CHUNK)])
    return 0

  lax.fori_loop(0, count, chunk_body, 0)


def _gather_kernel(context, batch):
  return pl.kernel(
      _gather_body,
      out_type=jax.ShapeDtypeStruct((N, H), jnp.float32),
      mesh=plsc.VectorSubcoreMesh(core_axis_name="c", subcore_axis_name="s"),
      scratch_types=[
          pltpu.VMEM((CHUNK,), jnp.int32),
          pltpu.VMEM((CHUNK, H), jnp.float32),
          pltpu.SemaphoreType.DMA,
      ],
  )(context, batch)


def kernel(x, batch, W1, b1, W2, b2, Wp, bp):
  batch = batch.astype(jnp.int32)
  context = _segment_context(
      x,
      batch.reshape(N, 1),
      W1,
      b1.reshape(1, H // 2),
      W2.reshape(1, H // 2),
      b2.reshape(1, 1),
      Wp,
      bp.reshape(1, H),
  )
  return _gather_kernel(context, batch)


# speedup vs baseline: 9.4690x; 2.4034x over previous
import jax
import jax.numpy as jnp
from jax import lax
from jax.experimental import pallas as pl
from jax.experimental.pallas import tpu as pltpu
from jax.experimental.pallas import tpu_sc as plsc

N = 320000
H = 128
G = 512
GW = 32
GPAD = G + GW

ROWS_PER_TILE = 2560
NUM_TILES = N // ROWS_PER_TILE


def _pool_body(x_ref, b_ref, w1_ref, b1_ref, w2r_ref, b2_ref, wp_ref, bp_ref,
               ctx_ref, acc_ex, acc_e):
  i = pl.program_id(0)
  T = ROWS_PER_TILE

  @pl.when(i == 0)
  def _init():
    acc_ex[...] = jnp.zeros_like(acc_ex)
    acc_e[...] = jnp.zeros_like(acc_e)

  xb = x_ref[...]
  btf = b_ref[...]
  w2r = w2r_ref[...]

  h = jnp.tanh(
      lax.dot_general(xb, w1_ref[...], (((1,), (1,)), ((), ())),
                      preferred_element_type=jnp.float32)
      + b1_ref[...])
  s_b = lax.dot_general(h, w2r, (((1,), (0,)), ((), ())),
                        preferred_element_type=jnp.float32)
  b2s = b2_ref[0, 0]
  shift = jnp.sum(jnp.abs(w2r[:, 0:1])) + jnp.abs(b2s)
  e_b = jnp.exp(s_b + (b2s - shift))

  bt32 = jnp.broadcast_to(btf, (T, GW))

  g_first = btf[0, 0].astype(jnp.int32)
  g_last = btf[T - 1, 0].astype(jnp.int32)
  n_win = (g_last - g_first) // GW + 1
  colf = lax.broadcasted_iota(jnp.int32, (1, GW), 1).astype(jnp.float32)
  ones = jnp.ones((T, GW), jnp.float32)

  def win_body(p, _):
    base = g_first + p * GW
    basef = base.astype(jnp.float32)
    oh_e = jnp.where(bt32 == basef + colf, e_b, 0.0)
    pex = lax.dot_general(oh_e, xb, (((0,), (0,)), ((), ())),
                          preferred_element_type=jnp.float32)
    pe = lax.dot_general(oh_e, ones, (((0,), (0,)), ((), ())),
                         preferred_element_type=jnp.float32)
    acc_ex[pl.ds(base, GW), :] += pex
    acc_e[pl.ds(base, GW), :] += pe
    return 0

  lax.fori_loop(0, n_win, win_body, 0)

  @pl.when(i == NUM_TILES - 1)
  def _finish():
    se = acc_e[0:G, 0:1]
    ge = acc_ex[0:G, :] / jnp.where(se > 0.0, se, 1.0)
    ctx = lax.dot_general(ge, wp_ref[...], (((1,), (1,)), ((), ())),
                          preferred_element_type=jnp.float32) + bp_ref[...]
    ctx_ref[...] = ctx


def _segment_context(x, batchf, W1, b1, W2r, b2, Wp, bp, interpret=False):
  T = ROWS_PER_TILE
  return pl.pallas_call(
      _pool_body,
      grid=(NUM_TILES,),
      in_specs=[
          pl.BlockSpec((T, H), lambda i: (i, 0)),
          pl.BlockSpec((T, 1), lambda i: (i, 0)),
          pl.BlockSpec((H // 2, H), lambda i: (0, 0)),
          pl.BlockSpec((1, H // 2), lambda i: (0, 0)),
          pl.BlockSpec((H // 2, GW), lambda i: (0, 0)),
          pl.BlockSpec((1, 1), lambda i: (0, 0)),
          pl.BlockSpec((H, H), lambda i: (0, 0)),
          pl.BlockSpec((1, H), lambda i: (0, 0)),
      ],
      out_specs=pl.BlockSpec((G, H), lambda i: (0, 0)),
      out_shape=jax.ShapeDtypeStruct((G, H), jnp.float32),
      scratch_shapes=[
          pltpu.VMEM((GPAD, H), jnp.float32),
          pltpu.VMEM((GPAD, GW), jnp.float32),
      ],
      compiler_params=pltpu.CompilerParams(
          dimension_semantics=("arbitrary",)),
      interpret=interpret,
  )(x, batchf, W1, b1, W2r, b2, Wp, bp)


CHUNK = 128
NUM_CHUNKS = N // CHUNK
NUM_WORKERS = 32
BASE_CHUNKS = NUM_CHUNKS // NUM_WORKERS
EXTRA = NUM_CHUNKS - BASE_CHUNKS * NUM_WORKERS
SLOTS = BASE_CHUNKS + 18
IDX_PAD_ROWS = NUM_WORKERS * BASE_CHUNKS + EXTRA + SLOTS
LANE = 16
COLG = H // LANE


def _gather_body(ctx_hbm, b2d_hbm, out_hbm,
                 idx_all, table, buf0, buf1, gsem0, gsem1, ssem0, ssem1):
  wid = lax.axis_index("s") * 2 + lax.axis_index("c")
  start = wid * BASE_CHUNKS + jnp.minimum(wid, EXTRA)
  count = BASE_CHUNKS + jnp.where(wid < EXTRA, 1, 0)

  pltpu.sync_copy(ctx_hbm, table)
  astart = (start // 8) * 8
  off = start - astart
  pltpu.sync_copy(b2d_hbm.at[pl.ds(astart, SLOTS)], idx_all)

  bufs = (buf0, buf1)
  gsems = (gsem0, gsem1)
  ssems = (ssem0, ssem1)

  def scatter_desc(c, b):
    base = (start + c) * CHUNK
    return pltpu.make_async_copy(
        bufs[b], out_hbm.at[pl.ds(base, CHUNK)], ssems[b])

  def replicate(buf, g):
    rows = [table[g, pl.ds(LANE * k, LANE)] for k in range(COLG)]

    def row_body(r, _):
      for k in range(COLG):
        buf[r, pl.ds(LANE * k, LANE)] = rows[k]
      return 0
    lax.fori_loop(0, CHUNK, row_body, 0)

  def chunk_body(c, carry):
    cur_g, parity, pend0, pend1 = carry

    mns = idx_all[off + c, pl.ds(0, LANE)][0]
    mxs = idx_all[off + c, pl.ds(CHUNK - LANE, LANE)][LANE - 1]
    uniform = mns == mxs
    reuse = uniform & (mns == cur_g)
    np_ = jnp.where(reuse, parity, 1 - parity)
    rebuild = jnp.logical_not(reuse)

    for b in range(2):
      @pl.when(rebuild & (np_ == b))
      def _rb(b=b):
        lax.fori_loop(0, (pend0, pend1)[b],
                      lambda j, _: (scatter_desc(c, b).wait(), 0)[1], 0)

        @pl.when(uniform)
        def _uni():
          replicate(bufs[b], mns)

        @pl.when(jnp.logical_not(uniform))
        def _mixed():
          pltpu.make_async_copy(ctx_hbm.at[idx_all.at[off + c]],
                                bufs[b], gsems[b]).start()
          pltpu.make_async_copy(ctx_hbm.at[idx_all.at[off + c]],
                                bufs[b], gsems[b]).wait()

      @pl.when(np_ == b)
      def _sc(b=b):
        scatter_desc(c, b).start()

    pend0 = jnp.where(np_ == 0, jnp.where(rebuild, 1, pend0 + 1), pend0)
    pend1 = jnp.where(np_ == 1, jnp.where(rebuild, 1, pend1 + 1), pend1)
    new_g = jnp.where(uniform, mns, -1)
    return new_g, np_, pend0, pend1

  init = (jnp.int32(-1), jnp.int32(0), jnp.int32(0), jnp.int32(0))
  _, _, pend0, pend1 = lax.fori_loop(0, count, chunk_body, init)

  lax.fori_loop(0, pend0, lambda j, _: (scatter_desc(0, 0).wait(), 0)[1], 0)
  lax.fori_loop(0, pend1, lambda j, _: (scatter_desc(0, 1).wait(), 0)[1], 0)


def _gather_kernel(context, batch2d):
  return pl.kernel(
      _gather_body,
      out_type=jax.ShapeDtypeStruct((N, H), jnp.float32),
      mesh=plsc.VectorSubcoreMesh(core_axis_name="c", subcore_axis_name="s"),
      scratch_types=[
          pltpu.VMEM((SLOTS, CHUNK), jnp.int32),
          pltpu.VMEM((G, H), jnp.float32),
          pltpu.VMEM((CHUNK, H), jnp.float32),
          pltpu.VMEM((CHUNK, H), jnp.float32),
          pltpu.SemaphoreType.DMA,
          pltpu.SemaphoreType.DMA,
          pltpu.SemaphoreType.DMA,
          pltpu.SemaphoreType.DMA,
      ],
  )(context, batch2d)


def kernel(x, batch, W1, b1, W2, b2, Wp, bp):
  batch = batch.astype(jnp.int32)
  context = _segment_context(
      x,
      batch.astype(jnp.float32).reshape(N, 1),
      W1,
      b1.reshape(1, H // 2),
      jnp.tile(W2.reshape(H // 2, 1), (1, GW)),
      b2.reshape(1, 1),
      Wp,
      bp.reshape(1, H),
  )
  batch2d = jnp.pad(batch.reshape(NUM_CHUNKS, CHUNK),
                    ((0, IDX_PAD_ROWS - NUM_CHUNKS), (0, 0)))
  return _gather_kernel(context, batch2d)


# speedup vs baseline: 12.3424x; 1.3035x over previous
import jax
import jax.numpy as jnp
from jax import lax
from jax.experimental import pallas as pl
from jax.experimental.pallas import tpu as pltpu
from jax.experimental.pallas import tpu_sc as plsc

N = 320000
H = 128
G = 512
GW = 32
GPAD = G + GW

ROWS_PER_TILE = 2560
NUM_TILES = N // ROWS_PER_TILE


def _pool_body(x_ref, b_ref, w1_ref, b1_ref, w2r_ref, b2_ref, wp_ref, bp_ref,
               ctx_ref, acc_ex, acc_e):
  i = pl.program_id(0)
  T = ROWS_PER_TILE

  @pl.when(i == 0)
  def _init():
    acc_ex[...] = jnp.zeros_like(acc_ex)
    acc_e[...] = jnp.zeros_like(acc_e)

  xb = x_ref[...]
  btf = b_ref[...]
  w2r = w2r_ref[...]

  h = jnp.tanh(
      lax.dot_general(xb, w1_ref[...], (((1,), (1,)), ((), ())),
                      preferred_element_type=jnp.float32)
      + b1_ref[...])
  s_b = lax.dot_general(h, w2r, (((1,), (0,)), ((), ())),
                        preferred_element_type=jnp.float32)
  b2s = b2_ref[0, 0]
  shift = jnp.sum(jnp.abs(w2r[:, 0:1])) + jnp.abs(b2s)
  e_b = jnp.exp(s_b + (b2s - shift))

  bt32 = jnp.broadcast_to(btf, (T, GW))

  g_first = btf[0, 0].astype(jnp.int32)
  g_last = btf[T - 1, 0].astype(jnp.int32)
  n_win = (g_last - g_first) // GW + 1
  colf = lax.broadcasted_iota(jnp.int32, (1, GW), 1).astype(jnp.float32)
  ones = jnp.ones((T, GW), jnp.float32)

  def win_body(p, _):
    base = g_first + p * GW
    basef = base.astype(jnp.float32)
    oh_e = jnp.where(bt32 == basef + colf, e_b, 0.0)
    pex = lax.dot_general(oh_e, xb, (((0,), (0,)), ((), ())),
                          preferred_element_type=jnp.float32)
    pe = lax.dot_general(oh_e, ones, (((0,), (0,)), ((), ())),
                         preferred_element_type=jnp.float32)
    acc_ex[pl.ds(base, GW), :] += pex
    acc_e[pl.ds(base, GW), :] += pe
    return 0

  lax.fori_loop(0, n_win, win_body, 0)

  @pl.when(i == NUM_TILES - 1)
  def _finish():
    se = acc_e[0:G, 0:1]
    ge = acc_ex[0:G, :] / jnp.where(se > 0.0, se, 1.0)
    ctx = lax.dot_general(ge, wp_ref[...], (((1,), (1,)), ((), ())),
                          preferred_element_type=jnp.float32) + bp_ref[...]
    ctx_ref[...] = ctx


def _segment_context(x, batchf, W1, b1, W2r, b2, Wp, bp, interpret=False):
  T = ROWS_PER_TILE
  return pl.pallas_call(
      _pool_body,
      grid=(NUM_TILES,),
      in_specs=[
          pl.BlockSpec((T, H), lambda i: (i, 0)),
          pl.BlockSpec((T, 1), lambda i: (i, 0)),
          pl.BlockSpec((H // 2, H), lambda i: (0, 0)),
          pl.BlockSpec((1, H // 2), lambda i: (0, 0)),
          pl.BlockSpec((H // 2, GW), lambda i: (0, 0)),
          pl.BlockSpec((1, 1), lambda i: (0, 0)),
          pl.BlockSpec((H, H), lambda i: (0, 0)),
          pl.BlockSpec((1, H), lambda i: (0, 0)),
      ],
      out_specs=pl.BlockSpec((G, H), lambda i: (0, 0)),
      out_shape=jax.ShapeDtypeStruct((G, H), jnp.float32),
      scratch_shapes=[
          pltpu.VMEM((GPAD, H), jnp.float32),
          pltpu.VMEM((GPAD, GW), jnp.float32),
      ],
      compiler_params=pltpu.CompilerParams(
          dimension_semantics=("arbitrary",)),
      interpret=interpret,
  )(x, batchf, W1, b1, W2r, b2, Wp, bp)


CHUNK = 128
NUM_CHUNKS = N // CHUNK
NUM_WORKERS = 32
BASE_CHUNKS = NUM_CHUNKS // NUM_WORKERS
EXTRA = NUM_CHUNKS - BASE_CHUNKS * NUM_WORKERS
SLOTS = BASE_CHUNKS + 18
IDX_PAD_ROWS = NUM_WORKERS * BASE_CHUNKS + EXTRA + SLOTS
LANE = 16
COLG = H // LANE


def _gather_body(ctx_hbm, b2d_hbm, out_hbm,
                 idx_all, table, buf0, buf1, ssem0, ssem1):
  wid = lax.axis_index("s") * 2 + lax.axis_index("c")
  start = wid * BASE_CHUNKS + jnp.minimum(wid, EXTRA)
  count = BASE_CHUNKS + jnp.where(wid < EXTRA, 1, 0)

  pltpu.sync_copy(ctx_hbm, table)
  astart = (start // 8) * 8
  off = start - astart
  pltpu.sync_copy(b2d_hbm.at[pl.ds(astart, SLOTS)], idx_all)

  bufs = (buf0, buf1)
  ssems = (ssem0, ssem1)

  def scatter_desc(c, b):
    base = (start + c) * CHUNK
    return pltpu.make_async_copy(
        bufs[b], out_hbm.at[pl.ds(base, CHUNK)], ssems[b])

  def replicate(buf, g):
    rows = [table[g, pl.ds(LANE * k, LANE)] for k in range(COLG)]

    def row_body(r, _):
      for k in range(COLG):
        buf[r, pl.ds(LANE * k, LANE)] = rows[k]
      return 0
    lax.fori_loop(0, CHUNK, row_body, 0)

  def perrow_fill(buf, c):
    def row_body(r, _):
      g = idx_all[off + c, pl.ds(r, LANE)][0]
      for k in range(COLG):
        buf[r, pl.ds(LANE * k, LANE)] = table[g, pl.ds(LANE * k, LANE)]
      return 0
    lax.fori_loop(0, CHUNK - LANE + 1, row_body, 0)

    tail = idx_all[off + c, pl.ds(CHUNK - LANE, LANE)]
    for l in range(1, LANE):
      g = tail[l]
      r = CHUNK - LANE + l
      for k in range(COLG):
        buf[r, pl.ds(LANE * k, LANE)] = table[g, pl.ds(LANE * k, LANE)]

  def chunk_body(c, carry):
    cur_g, pend0, pend1 = carry

    mns = idx_all[off + c, pl.ds(0, LANE)][0]
    mxs = idx_all[off + c, pl.ds(CHUNK - LANE, LANE)][LANE - 1]
    uniform = mns == mxs

    @pl.when(uniform & (mns != cur_g))
    def _rb_uni():
      lax.fori_loop(0, pend1,
                    lambda j, _: (scatter_desc(c, 1).wait(), 0)[1], 0)
      replicate(buf1, mns)

    @pl.when(jnp.logical_not(uniform))
    def _rb_mix():
      lax.fori_loop(0, pend0,
                    lambda j, _: (scatter_desc(c, 0).wait(), 0)[1], 0)
      perrow_fill(buf0, c)

    @pl.when(uniform)
    def _sc1():
      scatter_desc(c, 1).start()

    @pl.when(jnp.logical_not(uniform))
    def _sc0():
      scatter_desc(c, 0).start()

    pend0 = jnp.where(uniform, pend0, 1)
    pend1 = jnp.where(uniform,
                      jnp.where(mns != cur_g, 1, pend1 + 1), pend1)
    new_g = jnp.where(uniform, mns, cur_g)
    return new_g, pend0, pend1

  init = (jnp.int32(-1), jnp.int32(0), jnp.int32(0))
  _, pend0, pend1 = lax.fori_loop(0, count, chunk_body, init)

  lax.fori_loop(0, pend0, lambda j, _: (scatter_desc(0, 0).wait(), 0)[1], 0)
  lax.fori_loop(0, pend1, lambda j, _: (scatter_desc(0, 1).wait(), 0)[1], 0)


def _gather_kernel(context, batch2d):
  return pl.kernel(
      _gather_body,
      out_type=jax.ShapeDtypeStruct((N, H), jnp.float32),
      mesh=plsc.VectorSubcoreMesh(core_axis_name="c", subcore_axis_name="s"),
      scratch_types=[
          pltpu.VMEM((SLOTS, CHUNK), jnp.int32),
          pltpu.VMEM((G, H), jnp.float32),
          pltpu.VMEM((CHUNK, H), jnp.float32),
          pltpu.VMEM((CHUNK, H), jnp.float32),
          pltpu.SemaphoreType.DMA,
          pltpu.SemaphoreType.DMA,
      ],
  )(context, batch2d)


def kernel(x, batch, W1, b1, W2, b2, Wp, bp):
  batch = batch.astype(jnp.int32)
  context = _segment_context(
      x,
      batch.astype(jnp.float32).reshape(N, 1),
      W1,
      b1.reshape(1, H // 2),
      jnp.tile(W2.reshape(H // 2, 1), (1, GW)),
      b2.reshape(1, 1),
      Wp,
      bp.reshape(1, H),
  )
  batch2d = jnp.pad(batch.reshape(NUM_CHUNKS, CHUNK),
                    ((0, IDX_PAD_ROWS - NUM_CHUNKS), (0, 0)))
  return _gather_kernel(context, batch2d)
